# baseline (speedup 1.0000x reference)
import functools

import jax
import jax.numpy as jnp
import numpy as np
from jax import lax
from jax.experimental import pallas as pl
from jax.experimental.pallas import tpu as pltpu
from jax.experimental.pallas import tpu_sc as plsc

_MAX_FRAMES = 12
_TOPK = 64
_N_TOKENS = 197
_D = 512
_B = 16
_OUT_TOK = _TOPK + 1
_OUT_L = _MAX_FRAMES * _OUT_TOK
_L = _MAX_FRAMES * _N_TOKENS
_NW = 32
_NCHUNK = 4
_CW = 7
_C_OFF = (0, 7, 13, 19)
_C_LEN = (7, 6, 6, 6)
_SPAN = 25
_MAX_BASE = _OUT_L - _SPAN

_SEL = np.array([
    2, 3, 4, 5, 7, 16, 19, 29, 30, 31, 34, 35, 37, 39, 42, 44, 45, 56,
    58, 61, 63, 65, 67, 70, 72, 78, 82, 83, 85, 90, 99, 101, 102, 108,
    110, 111, 112, 114, 117, 121, 123, 129, 130, 139, 142, 144, 148, 152,
    153, 155, 156, 157, 163, 167, 174, 175, 176, 177, 178, 179, 183, 186,
    188, 189], dtype=np.int32)
_FRAME_ROWS = np.concatenate([[0], 1 + _SEL]).astype(np.int32)

_ROW_MAP = ((np.arange(_OUT_L) // _OUT_TOK) * _N_TOKENS
            + _FRAME_ROWS[np.arange(_OUT_L) % _OUT_TOK]).astype(np.int32)


def _build_index_table() -> np.ndarray:
    tbl = np.zeros((_NW, 8, 128), np.int32)
    for w in range(_NW):
        base = min(w * _OUT_L // _NW, _MAX_BASE)
        for c in range(_NCHUNK):
            s = base + _C_OFF[c]
            n = _C_LEN[c]
            tbl[w, c, :n] = _ROW_MAP[s:s + n]
    return tbl


_IDX_TBL = _build_index_table()


@functools.partial(
    pl.kernel,
    out_type=jax.ShapeDtypeStruct((_OUT_L, _B, _D), jnp.float32),
    mesh=plsc.VectorSubcoreMesh(core_axis_name="c", subcore_axis_name="s"),
    compiler_params=pltpu.CompilerParams(use_tc_tiling_on_sc=True),
    scratch_types=[
        pltpu.VMEM((8, 128), jnp.int32),
        pltpu.VMEM((_CW, _B, _D), jnp.float32),
        pltpu.VMEM((_CW, _B, _D), jnp.float32),
        pltpu.SemaphoreType.DMA,
        pltpu.SemaphoreType.DMA,
        pltpu.SemaphoreType.DMA,
        pltpu.SemaphoreType.DMA,
    ],
)
def _gather_tokens(x_hbm, idx_hbm, out_hbm, idx_v,
                   buf0, buf1, gs0, gs1, ws0, ws1):
    wid = lax.axis_index("s") * 2 + lax.axis_index("c")
    pltpu.sync_copy(idx_hbm.at[wid], idx_v)
    bufs = (buf0, buf1)
    gsems = (gs0, gs1)
    wsems = (ws0, ws1)

    base = lax.min(wid * _OUT_L // _NW, _MAX_BASE)

    def start_gather(c):
        return pltpu.async_copy(x_hbm.at[idx_v.at[c, pl.ds(0, _C_LEN[c])]],
                                bufs[c % 2].at[pl.ds(0, _C_LEN[c])],
                                gsems[c % 2])

    def start_store(c):
        return pltpu.async_copy(bufs[c % 2].at[pl.ds(0, _C_LEN[c])],
                                out_hbm.at[pl.ds(base + _C_OFF[c], _C_LEN[c])],
                                wsems[c % 2])

    gathers = [start_gather(0), start_gather(1)]
    writes = [None] * _NCHUNK
    for c in range(_NCHUNK):
        gathers[c % 2].wait()
        writes[c] = start_store(c)
        if c + 2 < _NCHUNK:
            writes[c].wait()
            gathers[c % 2] = start_gather(c + 2)
    writes[_NCHUNK - 2].wait()
    writes[_NCHUNK - 1].wait()


def kernel(x):
    xt = jnp.transpose(x, (1, 0, 2))
    out_t = _gather_tokens(xt, jnp.asarray(_IDX_TBL))
    return jnp.transpose(out_t, (1, 0, 2))
